# Initial kernel scaffold; baseline (speedup 1.0000x reference)
import jax
import jax.numpy as jnp
from jax.experimental import pallas as pl


def kernel(x, params):
    raise NotImplementedError("write your pallas kernel here")


# speedup vs baseline: 15.5798x; 15.5798x over previous
import functools

import jax
import jax.numpy as jnp
import numpy as np
from jax import lax
from jax.experimental import pallas as pl


def _fps_body(xs_ref, ys_ref, zs_ref, ox_ref, oy_ref, oz_ref, *, n_samples):
    x = xs_ref[...]
    y = ys_ref[...]
    z = zs_ref[...]
    B, N = x.shape
    iota = lax.broadcasted_iota(jnp.int32, (B, N), 1)
    iota_s = lax.broadcasted_iota(jnp.int32, (B, n_samples), 1)
    dist0 = jnp.full((B, N), 1e10, jnp.float32)
    far0 = jnp.zeros((B, 1), jnp.int32)
    acc0 = jnp.zeros((B, n_samples), jnp.float32)

    def body(i, carry):
        dist, far, ax, ay, az = carry
        oh = iota == far
        cx = jnp.sum(jnp.where(oh, x, 0.0), axis=1, keepdims=True)
        cy = jnp.sum(jnp.where(oh, y, 0.0), axis=1, keepdims=True)
        cz = jnp.sum(jnp.where(oh, z, 0.0), axis=1, keepdims=True)
        upd = iota_s == i
        ax = jnp.where(upd, cx, ax)
        ay = jnp.where(upd, cy, ay)
        az = jnp.where(upd, cz, az)
        dx = x - cx
        dy = y - cy
        dz = z - cz
        d = dx * dx + dy * dy + dz * dz
        dist = jnp.minimum(dist, d)
        m = jnp.max(dist, axis=1, keepdims=True)
        far = jnp.min(jnp.where(dist == m, iota, N), axis=1, keepdims=True)
        return dist, far, ax, ay, az

    _, _, ax, ay, az = lax.fori_loop(
        0, n_samples, body, (dist0, far0, acc0, acc0, acc0))
    ox_ref[...] = ax
    oy_ref[...] = ay
    oz_ref[...] = az


def _fps(xyz, n_samples):
    B, N, _ = xyz.shape
    xt = jnp.transpose(xyz, (0, 2, 1))
    outs = pl.pallas_call(
        functools.partial(_fps_body, n_samples=n_samples),
        out_shape=[jax.ShapeDtypeStruct((B, n_samples), jnp.float32)] * 3,
    )(xt[:, 0], xt[:, 1], xt[:, 2])
    return jnp.stack(outs, axis=-1)


def _cumsum_lanes(e):
    T, N = e.shape
    s = 1
    while s < N:
        shifted = jnp.concatenate(
            [jnp.zeros((T, s), jnp.float32), e[:, :-s]], axis=1)
        e = e + shifted
        s *= 2
    return e


def _make_sa_kernel(r2, K, KC, Tc, N, CF, nlayers):
    def body(c_ref, xt_ref, F_ref, *refs):
        o_ref = refs[-1]
        wrefs = refs[:-1]
        c = c_ref[0]
        xt = xt_ref[0]
        F = F_ref[0]
        aa = jnp.sum(c * c, axis=1, keepdims=True)
        bb = jnp.sum(xt * xt, axis=0, keepdims=True)
        ab = jnp.dot(c, xt, preferred_element_type=jnp.float32)
        d = jnp.maximum(aa + bb - 2.0 * ab, 0.0)
        msk = d <= r2
        e = _cumsum_lanes(msk.astype(jnp.float32))
        cnt = e[:, N - 1:N]
        em = jnp.where(msk, jnp.minimum(e, 33.0), 0.0)
        groups = []
        for kc in range(K // KC):
            kv = (lax.broadcasted_iota(jnp.int32, (1, KC, 1), 1)
                  + (kc * KC + 1)).astype(jnp.float32)
            O = (em[:, None, :] == kv).astype(jnp.float32)
            g = jnp.dot(O.reshape(Tc * KC, N), F,
                        preferred_element_type=jnp.float32)
            groups.append(g.reshape(Tc, KC, CF))
        grouped = jnp.concatenate(groups, axis=1)
        kr = lax.broadcasted_iota(
            jnp.int32, (Tc, K, 1), 1).astype(jnp.float32)
        cnt3 = cnt[:, :, None]
        g0 = jnp.where(cnt3 > 0.0, grouped[:, 0:1, :], F[0:1, :][None])
        grouped = jnp.where(kr < cnt3, grouped, g0)
        dx = grouped[..., :3] - c[:, None, :]
        h = jnp.concatenate([dx, grouped[..., 3:]], axis=-1)
        h = h.reshape(Tc * K, CF)
        for i in range(nlayers):
            W = wrefs[2 * i][...]
            b = wrefs[2 * i + 1][...]
            h = jnp.maximum(
                jnp.dot(h, W, preferred_element_type=jnp.float32) + b, 0.0)
        Cout = h.shape[-1]
        out = jnp.max(h.reshape(Tc, K, Cout), axis=1)
        o_ref[0] = out

    return body


def _sa_level(new_xyz, xyz, feats, mlp, radius, K, Tc, KC):
    B, N, _ = xyz.shape
    S = new_xyz.shape[1]
    CF = 3 + feats.shape[-1]
    F = jnp.concatenate([xyz, feats], axis=-1)
    xt = jnp.transpose(xyz, (0, 2, 1))
    wb = []
    for (W, b) in mlp:
        wb += [W, b.reshape(1, -1)]
    Cout = mlp[-1][0].shape[1]
    r2 = np.float32(radius * radius)
    in_specs = [
        pl.BlockSpec((1, Tc, 3), lambda b, s: (b, s, 0)),
        pl.BlockSpec((1, 3, N), lambda b, s: (b, 0, 0)),
        pl.BlockSpec((1, N, CF), lambda b, s: (b, 0, 0)),
    ] + [pl.BlockSpec(w.shape, functools.partial(
        lambda nd, b, s: (0,) * nd, w.ndim)) for w in wb]
    return pl.pallas_call(
        _make_sa_kernel(r2, K, KC, Tc, N, CF, len(mlp)),
        grid=(B, S // Tc),
        in_specs=in_specs,
        out_specs=pl.BlockSpec((1, Tc, Cout), lambda b, s: (b, s, 0)),
        out_shape=jax.ShapeDtypeStruct((B, S, Cout), jnp.float32),
    )(new_xyz, xt, F, *wb)


def _make_fp_kernel(Td, Ns, nrelu, has_head):
    def body(dx_ref, sxt_ref, sf_ref, df_ref, *refs):
        o_ref = refs[-1]
        wrefs = refs[:-1]
        cd = dx_ref[0]
        sxt = sxt_ref[0]
        SF = sf_ref[0]
        DF = df_ref[0]
        aa = jnp.sum(cd * cd, axis=1, keepdims=True)
        bb = jnp.sum(sxt * sxt, axis=0, keepdims=True)
        ab = jnp.dot(cd, sxt, preferred_element_type=jnp.float32)
        d = jnp.maximum(aa + bb - 2.0 * ab, 0.0)
        iota = lax.broadcasted_iota(jnp.int32, (Td, Ns), 1)
        ohs, ws = [], []
        for _ in range(3):
            mval = jnp.min(d, axis=1, keepdims=True)
            idx = jnp.min(jnp.where(d == mval, iota, Ns), axis=1,
                          keepdims=True)
            oh = iota == idx
            ws.append(1.0 / (mval + 1e-8))
            ohs.append(oh)
            d = jnp.where(oh, jnp.inf, d)
        wsum = ws[0] + ws[1] + ws[2]
        Wm = (ohs[0].astype(jnp.float32) * (ws[0] / wsum)
              + ohs[1].astype(jnp.float32) * (ws[1] / wsum)
              + ohs[2].astype(jnp.float32) * (ws[2] / wsum))
        interp = jnp.dot(Wm, SF, preferred_element_type=jnp.float32)
        h = jnp.concatenate([interp, DF], axis=-1)
        for i in range(nrelu):
            W = wrefs[2 * i][...]
            b = wrefs[2 * i + 1][...]
            h = jnp.maximum(
                jnp.dot(h, W, preferred_element_type=jnp.float32) + b, 0.0)
        if has_head:
            W = wrefs[2 * nrelu][...]
            b = wrefs[2 * nrelu + 1][...]
            h = jnp.dot(h, W, preferred_element_type=jnp.float32) + b
        o_ref[0] = h

    return body


def _fp_level(dst_xyz, dst_feats, src_xyz, src_feats, mlp, Td, head=None):
    B, Nd, _ = dst_xyz.shape
    Ns = src_xyz.shape[1]
    Cs = src_feats.shape[-1]
    Cd = dst_feats.shape[-1]
    sxt = jnp.transpose(src_xyz, (0, 2, 1))
    wb = []
    for (W, b) in mlp:
        wb += [W, b.reshape(1, -1)]
    Cout = mlp[-1][0].shape[1]
    if head is not None:
        Wh, bh = head
        wb += [Wh, bh.reshape(1, -1)]
        Cout = Wh.shape[1]
    in_specs = [
        pl.BlockSpec((1, Td, 3), lambda b, s: (b, s, 0)),
        pl.BlockSpec((1, 3, Ns), lambda b, s: (b, 0, 0)),
        pl.BlockSpec((1, Ns, Cs), lambda b, s: (b, 0, 0)),
        pl.BlockSpec((1, Td, Cd), lambda b, s: (b, s, 0)),
    ] + [pl.BlockSpec(w.shape, functools.partial(
        lambda nd, b, s: (0,) * nd, w.ndim)) for w in wb]
    return pl.pallas_call(
        _make_fp_kernel(Td, Ns, len(mlp), head is not None),
        grid=(B, Nd // Td),
        in_specs=in_specs,
        out_specs=pl.BlockSpec((1, Td, Cout), lambda b, s: (b, s, 0)),
        out_shape=jax.ShapeDtypeStruct((B, Nd, Cout), jnp.float32),
    )(dst_xyz, sxt, src_feats, dst_feats, *wb)


def kernel(x, params):
    xyz = jnp.transpose(x[:, :3, :], (0, 2, 1))
    feats = jnp.transpose(x[:, 3:, :], (0, 2, 1))
    sa = params['sa']
    fp = params['fp']
    Wh, bh = params['head']

    x0, f0 = xyz, feats
    x1 = _fps(x0, 1024)
    f1 = _sa_level(x1, x0, f0, sa[0], 0.4, 32, Tc=32, KC=8)
    x2 = _fps(x1, 256)
    f2 = _sa_level(x2, x1, f1, sa[1], 0.8, 32, Tc=32, KC=32)
    x3 = _fps(x2, 64)
    f3 = _sa_level(x3, x2, f2, sa[2], 1.6, 32, Tc=64, KC=32)
    x4 = _fps(x3, 16)
    f4 = _sa_level(x4, x3, f3, sa[3], 3.2, 32, Tc=16, KC=32)

    g1 = _fp_level(x3, f3, x4, f4, fp[0], Td=64)
    g2 = _fp_level(x2, f2, x3, g1, fp[1], Td=256)
    g3 = _fp_level(x1, f1, x2, g2, fp[2], Td=1024)
    g4 = _fp_level(x0, f0, x1, g3, fp[3], Td=1024, head=(Wh, bh))
    return jnp.transpose(g4, (0, 2, 1))
